# baseline (speedup 1.0000x reference)
import jax
from jax import lax
import jax.numpy as jnp
from jax.experimental import pallas as pl
from jax.experimental.pallas import tpu as pltpu
from jax.experimental.pallas import tpu_sc as plsc

_W = 128
_NBUF = 4
_NC = 2
_NS = 16
_NT = _NC * _NS


def kernel(x, embedding_weight):
    B, S = x.shape
    V, D = embedding_weight.shape
    n = B * S
    per_tile = n // _NT
    nsteps = per_tile // _W
    idx = x.reshape(n).astype(jnp.int32)

    mesh = plsc.VectorSubcoreMesh(
        core_axis_name="core", subcore_axis_name="subcore"
    )

    vmem_rows = pltpu.VMEM((_W, D), jnp.float32)

    @pl.kernel(
        out_type=jax.ShapeDtypeStruct((n, D), embedding_weight.dtype),
        mesh=mesh,
        scratch_types=[
            pltpu.VMEM((per_tile,), jnp.int32),
            vmem_rows, vmem_rows, vmem_rows, vmem_rows,
            pltpu.SemaphoreType.DMA, pltpu.SemaphoreType.DMA,
            pltpu.SemaphoreType.DMA, pltpu.SemaphoreType.DMA,
            pltpu.SemaphoreType.DMA, pltpu.SemaphoreType.DMA,
            pltpu.SemaphoreType.DMA, pltpu.SemaphoreType.DMA,
        ],
    )
    def gather_kernel(table_hbm, idx_hbm, out_hbm, idx_v,
                      buf0, buf1, buf2, buf3,
                      g0, g1, g2, g3, w0, w1, w2, w3):
        tile = lax.axis_index("subcore") * _NC + lax.axis_index("core")
        base = tile * per_tile
        pltpu.sync_copy(idx_hbm.at[pl.ds(base, per_tile)], idx_v)

        bufs = (buf0, buf1, buf2, buf3)
        gsems = (g0, g1, g2, g3)
        wsems = (w0, w1, w2, w3)

        def start_gather(st, b):
            pltpu.async_copy(
                table_hbm.at[idx_v.at[pl.ds(st * _W, _W)]], bufs[b], gsems[b]
            )

        start_gather(0, 0)

        @pl.loop(0, nsteps, step=_NBUF)
        def _(s):
            for b in range(_NBUF):
                st = s + b
                bn = (b + 1) % _NBUF

                @pl.when(st + 1 < nsteps)
                def _():
                    @pl.when(st + 1 >= _NBUF)
                    def _():
                        pltpu.make_async_copy(
                            bufs[bn],
                            out_hbm.at[pl.ds(base + (st + 1) * _W, _W)],
                            wsems[bn],
                        ).wait()

                    start_gather(st + 1, bn)

                pltpu.make_async_copy(
                    table_hbm.at[idx_v.at[pl.ds(st * _W, _W)]],
                    bufs[b],
                    gsems[b],
                ).wait()
                pltpu.async_copy(
                    bufs[b], out_hbm.at[pl.ds(base + st * _W, _W)], wsems[b]
                )

        for b in range(_NBUF):
            st = nsteps - _NBUF + b
            pltpu.make_async_copy(
                bufs[b], out_hbm.at[pl.ds(base + st * _W, _W)], wsems[b]
            ).wait()

    out = gather_kernel(embedding_weight, idx)
    return out.reshape(B, S, D)
